# Initial kernel scaffold; baseline (speedup 1.0000x reference)
import functools

import jax
import jax.numpy as jnp
from jax import lax
from jax.experimental import pallas as pl
from jax.experimental.pallas import tpu as pltpu
from jax.experimental.pallas import tpu_sc as plsc

L = 16


def _make_sc_kernel(N, Npad, chunk, Dpad, nx, ny, nz, NC, NS):
    NW = NC * NS
    niter = chunk // L
    mesh = plsc.VectorSubcoreMesh(core_axis_name="c", subcore_axis_name="s")

    @functools.partial(
        pl.kernel,
        mesh=mesh,
        out_type=[
            jax.ShapeDtypeStruct((Npad,), jnp.float32),
            jax.ShapeDtypeStruct((NW, L), jnp.float32),
        ],
        scratch_types=[
            pltpu.VMEM((6 * chunk,), jnp.float32),
            pltpu.VMEM((Dpad,), jnp.float32),
            pltpu.VMEM((6, L), jnp.float32),
            pltpu.VMEM((chunk,), jnp.float32),
            pltpu.VMEM((L,), jnp.float32),
        ],
    )
    def sc_kernel(coords_hbm, d_hbm, par_hbm, out_hbm, sums_hbm,
                  coords_v, d_v, par_v, out_v, sum_v):
        wid = lax.axis_index("s") * NC + lax.axis_index("c")
        base = wid * chunk
        pltpu.sync_copy(d_hbm, d_v)
        for c in range(6):
            pltpu.sync_copy(coords_hbm.at[pl.ds(c * Npad + base, chunk)],
                            coords_v.at[pl.ds(c * chunk, chunk)])
        pltpu.sync_copy(par_hbm, par_v)

        sxv = par_v[0]
        syv = par_v[1]
        szv = par_v[2]
        oxv = par_v[3]
        oyv = par_v[4]
        ozv = par_v[5]
        lane = lax.iota(jnp.int32, 16)
        snx = ny * nz
        sny = nz

        def body(i, acc):
            o = i * L
            yx = coords_v[pl.ds(o, L)] + coords_v[pl.ds(3 * chunk + o, L)]
            yy = (coords_v[pl.ds(chunk + o, L)]
                  + coords_v[pl.ds(4 * chunk + o, L)])
            yz = (coords_v[pl.ds(2 * chunk + o, L)]
                  + coords_v[pl.ds(5 * chunk + o, L)])
            gx = jnp.minimum(jnp.maximum(yx * sxv + oxv, 0.0), float(nx - 1))
            gy = jnp.minimum(jnp.maximum(yy * syv + oyv, 0.0), float(ny - 1))
            gz = jnp.minimum(jnp.maximum(yz * szv + ozv, 0.0), float(nz - 1))
            x0 = gx.astype(jnp.int32)
            y0 = gy.astype(jnp.int32)
            z0 = gz.astype(jnp.int32)
            wx = gx - x0.astype(jnp.float32)
            wy = gy - y0.astype(jnp.float32)
            wz = gz - z0.astype(jnp.float32)
            x1 = jnp.minimum(x0 + 1, nx - 1)
            y1 = jnp.minimum(y0 + 1, ny - 1)
            z1 = jnp.minimum(z0 + 1, nz - 1)
            ix0 = x0 * snx
            ix1 = x1 * snx
            iy0 = y0 * sny
            iy1 = y1 * sny
            a00 = ix0 + iy0
            a01 = ix0 + iy1
            a10 = ix1 + iy0
            a11 = ix1 + iy1
            c000 = plsc.load_gather(d_v, [a00 + z0])
            c001 = plsc.load_gather(d_v, [a00 + z1])
            c010 = plsc.load_gather(d_v, [a01 + z0])
            c011 = plsc.load_gather(d_v, [a01 + z1])
            c100 = plsc.load_gather(d_v, [a10 + z0])
            c101 = plsc.load_gather(d_v, [a10 + z1])
            c110 = plsc.load_gather(d_v, [a11 + z0])
            c111 = plsc.load_gather(d_v, [a11 + z1])
            c00 = c000 + wz * (c001 - c000)
            c01 = c010 + wz * (c011 - c010)
            c10 = c100 + wz * (c101 - c100)
            c11 = c110 + wz * (c111 - c110)
            c0 = c00 + wy * (c01 - c00)
            c1 = c10 + wy * (c11 - c10)
            val = c0 + wx * (c1 - c0)
            out_v[pl.ds(o, L)] = val
            valid = (base + o + lane) < N
            return acc + jnp.where(valid, val, 0.0)

        acc = lax.fori_loop(0, niter, body, jnp.zeros((L,), jnp.float32))
        sum_v[...] = acc
        pltpu.sync_copy(out_v, out_hbm.at[pl.ds(base, chunk)])
        pltpu.sync_copy(sum_v, sums_hbm.at[wid])

    return sc_kernel


def kernel(pc1, flow, D, grid_lo, grid_hi):
    N = pc1.shape[1]
    nx, ny, nz = D.shape
    info = plsc.get_sparse_core_info()
    NC, NS = info.num_cores, info.num_subcores
    NW = NC * NS
    chunk = (-(-N // NW) + L - 1) // L * L
    Npad = chunk * NW

    coords = jnp.concatenate(
        [pc1[0].T.reshape(3, N), flow[0].T.reshape(3, N)], axis=0)
    coords = jnp.pad(coords, ((0, 0), (0, Npad - N))).reshape(-1)

    Dlen = nx * ny * nz
    Dpad = -(-Dlen // 8) * 8
    d_flat = jnp.pad(D.reshape(-1), (0, Dpad - Dlen))

    span = grid_hi.astype(jnp.float32) - grid_lo.astype(jnp.float32)
    dims = jnp.array([nx - 1, ny - 1, nz - 1], jnp.float32)
    scale = dims / span
    offset = -grid_lo.astype(jnp.float32) * scale
    params = jnp.broadcast_to(
        jnp.concatenate([scale, offset])[:, None], (6, L)
    ).astype(jnp.float32)

    sc = _make_sc_kernel(N, Npad, chunk, Dpad, nx, ny, nz, NC, NS)
    out_pad, sums = sc(coords, d_flat, params)
    dt_loss = out_pad[:N]
    mean = sums.sum() / jnp.float32(N)
    return (mean, dt_loss)


# speedup vs baseline: 326.1896x; 326.1896x over previous
import functools

import jax
import jax.numpy as jnp
from jax import lax
from jax.experimental import pallas as pl
from jax.experimental.pallas import tpu as pltpu
from jax.experimental.pallas import tpu_sc as plsc

L = 16


def _make_sc_kernel(N, Npad, chunk, Dpad, nx, ny, nz, NC, NS):
    NW = NC * NS
    niter = chunk // L
    mesh = plsc.VectorSubcoreMesh(core_axis_name="c", subcore_axis_name="s")

    @functools.partial(
        pl.kernel,
        mesh=mesh,
        compiler_params=pltpu.CompilerParams(needs_layout_passes=False),
        out_type=[
            jax.ShapeDtypeStruct((Npad,), jnp.float32),
            jax.ShapeDtypeStruct((NW, L), jnp.float32),
        ],
        scratch_types=[
            pltpu.VMEM((6 * chunk,), jnp.float32),
            pltpu.VMEM((Dpad,), jnp.float32),
            pltpu.VMEM((6, L), jnp.float32),
            pltpu.VMEM((chunk,), jnp.float32),
            pltpu.VMEM((L,), jnp.float32),
        ],
    )
    def sc_kernel(coords_hbm, d_hbm, par_hbm, out_hbm, sums_hbm,
                  coords_v, d_v, par_v, out_v, sum_v):
        wid = lax.axis_index("s") * NC + lax.axis_index("c")
        base = wid * chunk
        pltpu.sync_copy(d_hbm, d_v)
        for c in range(6):
            pltpu.sync_copy(coords_hbm.at[pl.ds(c * Npad + base, chunk)],
                            coords_v.at[pl.ds(c * chunk, chunk)])
        pltpu.sync_copy(par_hbm, par_v)

        sxv = par_v[0]
        syv = par_v[1]
        szv = par_v[2]
        oxv = par_v[3]
        oyv = par_v[4]
        ozv = par_v[5]
        lane = lax.iota(jnp.int32, 16)
        snx = ny * nz
        sny = nz

        def body(i, acc):
            o = i * L
            yx = coords_v[pl.ds(o, L)] + coords_v[pl.ds(3 * chunk + o, L)]
            yy = (coords_v[pl.ds(chunk + o, L)]
                  + coords_v[pl.ds(4 * chunk + o, L)])
            yz = (coords_v[pl.ds(2 * chunk + o, L)]
                  + coords_v[pl.ds(5 * chunk + o, L)])
            gx = jnp.minimum(jnp.maximum(yx * sxv + oxv, 0.0), float(nx - 1))
            gy = jnp.minimum(jnp.maximum(yy * syv + oyv, 0.0), float(ny - 1))
            gz = jnp.minimum(jnp.maximum(yz * szv + ozv, 0.0), float(nz - 1))
            x0 = gx.astype(jnp.int32)
            y0 = gy.astype(jnp.int32)
            z0 = gz.astype(jnp.int32)
            wx = gx - x0.astype(jnp.float32)
            wy = gy - y0.astype(jnp.float32)
            wz = gz - z0.astype(jnp.float32)
            x1 = jnp.minimum(x0 + 1, nx - 1)
            y1 = jnp.minimum(y0 + 1, ny - 1)
            z1 = jnp.minimum(z0 + 1, nz - 1)
            ix0 = x0 * snx
            ix1 = x1 * snx
            iy0 = y0 * sny
            iy1 = y1 * sny
            a00 = ix0 + iy0
            a01 = ix0 + iy1
            a10 = ix1 + iy0
            a11 = ix1 + iy1
            c000 = plsc.load_gather(d_v, [a00 + z0])
            c001 = plsc.load_gather(d_v, [a00 + z1])
            c010 = plsc.load_gather(d_v, [a01 + z0])
            c011 = plsc.load_gather(d_v, [a01 + z1])
            c100 = plsc.load_gather(d_v, [a10 + z0])
            c101 = plsc.load_gather(d_v, [a10 + z1])
            c110 = plsc.load_gather(d_v, [a11 + z0])
            c111 = plsc.load_gather(d_v, [a11 + z1])
            c00 = c000 + wz * (c001 - c000)
            c01 = c010 + wz * (c011 - c010)
            c10 = c100 + wz * (c101 - c100)
            c11 = c110 + wz * (c111 - c110)
            c0 = c00 + wy * (c01 - c00)
            c1 = c10 + wy * (c11 - c10)
            val = c0 + wx * (c1 - c0)
            out_v[pl.ds(o, L)] = val
            valid = (base + o + lane) < N
            return acc + jnp.where(valid, val, 0.0)

        acc = lax.fori_loop(0, niter, body, jnp.zeros((L,), jnp.float32))
        sum_v[...] = acc
        pltpu.sync_copy(out_v, out_hbm.at[pl.ds(base, chunk)])
        pltpu.sync_copy(sum_v, sums_hbm.at[wid])

    return sc_kernel


def kernel(pc1, flow, D, grid_lo, grid_hi):
    N = pc1.shape[1]
    nx, ny, nz = D.shape
    info = plsc.get_sparse_core_info()
    NC, NS = info.num_cores, info.num_subcores
    NW = NC * NS
    chunk = (-(-N // NW) + L - 1) // L * L
    Npad = chunk * NW

    coords = jnp.concatenate(
        [pc1[0].T.reshape(3, N), flow[0].T.reshape(3, N)], axis=0)
    coords = jnp.pad(coords, ((0, 0), (0, Npad - N))).reshape(-1)

    Dlen = nx * ny * nz
    Dpad = -(-Dlen // 8) * 8
    d_flat = jnp.pad(D.reshape(-1), (0, Dpad - Dlen))

    span = grid_hi.astype(jnp.float32) - grid_lo.astype(jnp.float32)
    dims = jnp.array([nx - 1, ny - 1, nz - 1], jnp.float32)
    scale = dims / span
    offset = -grid_lo.astype(jnp.float32) * scale
    params = jnp.broadcast_to(
        jnp.concatenate([scale, offset])[:, None], (6, L)
    ).astype(jnp.float32)

    sc = _make_sc_kernel(N, Npad, chunk, Dpad, nx, ny, nz, NC, NS)
    out_pad, sums = sc(coords, d_flat, params)
    dt_loss = out_pad[:N]
    mean = sums.sum() / jnp.float32(N)
    return (mean, dt_loss)
